# baseline (speedup 1.0000x reference)
import functools

import jax
import jax.numpy as jnp
from jax import lax
from jax.experimental import pallas as pl
from jax.experimental.pallas import tpu as pltpu
from jax.experimental.pallas import tpu_sc as plsc

DIM = 512
NSL = 8
ROWS = 6
BATCH = 16384
NC, NSUB, L = 2, 16, 16
NW = NC * NSUB
BPW = BATCH // NW
CH = 32
NCHUNK = BPW // CH
NB = 3


@functools.partial(
    pl.kernel,
    out_type=jax.ShapeDtypeStruct((BATCH, 1, DIM), jnp.float32),
    mesh=plsc.VectorSubcoreMesh(core_axis_name="c", subcore_axis_name="s"),
    scratch_types=[
        pltpu.VMEM((BPW,), jnp.int32),
        pltpu.VMEM((ROWS, NSL * DIM // 2), jnp.float32),
        pltpu.VMEM((ROWS, 1, DIM), jnp.float32),
        pltpu.VMEM((CH, 1, DIM), jnp.float32),
        pltpu.VMEM((CH, 1, DIM), jnp.float32),
        pltpu.VMEM((CH, 1, DIM), jnp.float32),
        pltpu.VMEM((CH, 1, DIM), jnp.float32),
        pltpu.VMEM((CH, 1, DIM), jnp.float32),
        pltpu.VMEM((CH, 1, DIM), jnp.float32),
        pltpu.SemaphoreType.DMA,
        pltpu.SemaphoreType.DMA,
        pltpu.SemaphoreType.DMA,
        pltpu.SemaphoreType.DMA,
        pltpu.SemaphoreType.DMA,
        pltpu.SemaphoreType.DMA,
    ],
)
def _walk_embed(z_hbm, idx_hbm, wt_hbm, out_hbm,
                idx_v, wtv, wsv, zb0, zb1, zb2, ob0, ob1, ob2,
                zs0, zs1, zs2, os0, os1, os2):
    wid = lax.axis_index("s") * NC + lax.axis_index("c")
    base = wid * BPW

    zb, ob = (zb0, zb1, zb2), (ob0, ob1, ob2)
    zs, osm = (zs0, zs1, zs2), (os0, os1, os2)

    def start_in(c, b):
        pltpu.async_copy(z_hbm.at[pl.ds(base + c * CH, CH)], zb[b], zs[b])

    for b in range(NB):
        start_in(b, b)
    pltpu.sync_copy(idx_hbm.at[pl.ds(base, BPW)], idx_v)

    for h in range(2):
        pltpu.sync_copy(wt_hbm.at[h], wtv)
        for r in range(ROWS):

            @plsc.parallel_loop(0, DIM // L, 1, unroll=2)
            def _(v):
                o = v * L
                acc = wtv[r, pl.ds(o, L)]
                for i in range(1, NSL // 2):
                    acc = acc + wtv[r, pl.ds(i * DIM + o, L)]
                if h == 0:
                    wsv[r, 0, pl.ds(o, L)] = acc
                else:
                    wsv[r, 0, pl.ds(o, L)] = wsv[r, 0, pl.ds(o, L)] + acc

    def do_chunk(c, b, may_wait_out, may_prefetch):
        row0 = base + c * CH
        pltpu.make_async_copy(z_hbm.at[pl.ds(row0, CH)], zb[b], zs[b]).wait()

        if may_wait_out:

            @pl.when(c >= NB)
            def _():
                pltpu.make_async_copy(
                    ob[b], out_hbm.at[pl.ds(row0, CH)], osm[b]).wait()

        for g in range(CH // L):
            idxv = idx_v[pl.ds(c * CH + g * L, L)]
            svals = [idxv[j] for j in range(L)]

            @plsc.parallel_loop(0, DIM // L, 1, unroll=2)
            def _(v):
                o = v * L
                for j in range(L):
                    r = g * L + j
                    ob[b][r, 0, pl.ds(o, L)] = (
                        zb[b][r, 0, pl.ds(o, L)]
                        + wsv[svals[j], 0, pl.ds(o, L)])

        pltpu.async_copy(ob[b], out_hbm.at[pl.ds(row0, CH)], osm[b])

        if may_prefetch:

            @pl.when(c + NB < NCHUNK)
            def _():
                start_in(c + NB, b)

    def trip(it, carry):
        for b in range(NB):
            do_chunk(it * NB + b, b, True, True)
        return carry

    lax.fori_loop(0, NCHUNK // NB, trip, 0)
    do_chunk(NCHUNK - 1, (NCHUNK - 1) % NB, True, False)

    for c in range(NCHUNK - NB, NCHUNK):
        b = c % NB
        row0 = base + c * CH
        pltpu.make_async_copy(ob[b], out_hbm.at[pl.ds(row0, CH)], osm[b]).wait()


def kernel(z, w, index_, alpha=1):
    wt = jnp.transpose(w.reshape(ROWS, DIM, NSL), (2, 0, 1))
    wt = wt.reshape(2, NSL // 2, ROWS, DIM).transpose(0, 2, 1, 3)
    wt = wt.reshape(2, ROWS, NSL // 2 * DIM)
    return _walk_embed(z, index_, wt)
